# Initial kernel scaffold; baseline (speedup 1.0000x reference)
import jax
import jax.numpy as jnp
from jax.experimental import pallas as pl


def kernel(adj, inputs, neigh_feats, W_neigh0, W_lin0, W_neigh1, W_lin1, W_clf, b_clf):
    raise NotImplementedError("write your pallas kernel here")


# speedup vs baseline: 1.2696x; 1.2696x over previous
import functools

import jax
import jax.numpy as jnp
from jax.experimental import pallas as pl

N = 8192
D = 256
H = 256
C = 64

BM_BIG = 256
BM_SMALL = 1024

_bf16 = jnp.bfloat16
_f32 = jnp.float32


def _neigh_lin_kernel(x_ref, w_ref, o_ref):
    o_ref[...] = jnp.dot(
        x_ref[...].astype(_bf16), w_ref[...], preferred_element_type=_f32
    ).astype(_bf16)


def _agg_deg_kernel(adj_ref, h_ref, agg_ref, deg_ref):
    a = adj_ref[...]
    deg_ref[...] = jnp.sum(a, axis=1, keepdims=True) + 1.0
    agg_ref[...] = jnp.dot(
        a.astype(_bf16), h_ref[...], preferred_element_type=_f32
    )


def _agg_kernel(adj_ref, h_ref, agg_ref):
    agg_ref[...] = jnp.dot(
        adj_ref[...].astype(_bf16), h_ref[...], preferred_element_type=_f32
    )


def _mid_kernel(x_ref, agg_ref, deg_ref, wa_ref, wb_ref, wn_ref,
                h1f_ref, h1_ref):
    hn = (agg_ref[...] / deg_ref[...]).astype(_bf16)
    z = jnp.dot(x_ref[...].astype(_bf16), wa_ref[...],
                preferred_element_type=_f32)
    z += jnp.dot(hn, wb_ref[...], preferred_element_type=_f32)
    z = jnp.maximum(z, 0.0)
    n = jnp.sqrt(jnp.sum(z * z, axis=1, keepdims=True))
    zn = z / jnp.maximum(n, 1e-12)
    h1f_ref[...] = zn.astype(_bf16)
    h1_ref[...] = jnp.dot(zn.astype(_bf16), wn_ref[...],
                          preferred_element_type=_f32).astype(_bf16)


def _final_kernel(h1f_ref, agg_ref, deg_ref, wa_ref, wb_ref, wc_ref, b_ref,
                  out_ref):
    hn = (agg_ref[...] / deg_ref[...]).astype(_bf16)
    z = jnp.dot(h1f_ref[...], wa_ref[...], preferred_element_type=_f32)
    z += jnp.dot(hn, wb_ref[...], preferred_element_type=_f32)
    n = jnp.sqrt(jnp.sum(z * z, axis=1, keepdims=True))
    zn = (z / jnp.maximum(n, 1e-12)).astype(_bf16)
    out_ref[...] = jnp.dot(zn, wc_ref[...],
                           preferred_element_type=_f32) + b_ref[...]


def _row_spec(bm, cols):
    return pl.BlockSpec((bm, cols), lambda i: (i, 0))


def _full_spec(rows, cols):
    return pl.BlockSpec((rows, cols), lambda i: (0, 0))


@functools.partial(jax.jit, static_argnames=())
def kernel(adj, inputs, neigh_feats, W_neigh0, W_lin0, W_neigh1, W_lin1,
           W_clf, b_clf):
    del neigh_feats
    grid_big = (N // BM_BIG,)
    grid_small = (N // BM_SMALL,)

    wn0 = W_neigh0.T.astype(_bf16)
    wl0a = W_lin0[:, :D].T.astype(_bf16)
    wl0b = W_lin0[:, D:].T.astype(_bf16)
    wn1 = W_neigh1.T.astype(_bf16)
    wl1a = W_lin1[:, :H].T.astype(_bf16)
    wl1b = W_lin1[:, H:].T.astype(_bf16)
    wc = W_clf.T.astype(_bf16)
    bc = b_clf.reshape(1, C)

    h0 = pl.pallas_call(
        _neigh_lin_kernel,
        grid=grid_small,
        in_specs=[_row_spec(BM_SMALL, D), _full_spec(D, D)],
        out_specs=_row_spec(BM_SMALL, D),
        out_shape=jax.ShapeDtypeStruct((N, D), _bf16),
    )(inputs, wn0)

    agg0, deg = pl.pallas_call(
        _agg_deg_kernel,
        grid=grid_big,
        in_specs=[_row_spec(BM_BIG, N), _full_spec(N, D)],
        out_specs=[_row_spec(BM_BIG, D), _row_spec(BM_BIG, 1)],
        out_shape=[jax.ShapeDtypeStruct((N, D), _f32),
                   jax.ShapeDtypeStruct((N, 1), _f32)],
    )(adj, h0)

    h1f, h1 = pl.pallas_call(
        _mid_kernel,
        grid=grid_small,
        in_specs=[_row_spec(BM_SMALL, D), _row_spec(BM_SMALL, D),
                  _row_spec(BM_SMALL, 1), _full_spec(D, H),
                  _full_spec(D, H), _full_spec(H, H)],
        out_specs=[_row_spec(BM_SMALL, H), _row_spec(BM_SMALL, H)],
        out_shape=[jax.ShapeDtypeStruct((N, H), _bf16),
                   jax.ShapeDtypeStruct((N, H), _bf16)],
    )(inputs, agg0, deg, wl0a, wl0b, wn1)

    agg1 = pl.pallas_call(
        _agg_kernel,
        grid=grid_big,
        in_specs=[_row_spec(BM_BIG, N), _full_spec(N, H)],
        out_specs=_row_spec(BM_BIG, H),
        out_shape=jax.ShapeDtypeStruct((N, H), _f32),
    )(adj, h1)

    out = pl.pallas_call(
        _final_kernel,
        grid=grid_small,
        in_specs=[_row_spec(BM_SMALL, H), _row_spec(BM_SMALL, H),
                  _row_spec(BM_SMALL, 1), _full_spec(H, H),
                  _full_spec(H, H), _full_spec(H, C), _full_spec(1, C)],
        out_specs=_row_spec(BM_SMALL, C),
        out_shape=jax.ShapeDtypeStruct((N, C), _f32),
    )(h1f, agg1, deg, wl1a, wl1b, wc, bc)

    return out
